# baseline (speedup 1.0000x reference)
import functools

import jax
import jax.numpy as jnp
from jax import lax
from jax.experimental import pallas as pl
from jax.experimental.pallas import tpu as pltpu
from jax.experimental.pallas import tpu_sc as plsc

_NUM_GRAPHS = 16
_NC = 2
_NS = 16
_NW = _NC * _NS
_L = 16
_NCOMP = 6
_ACC_ROWS = _NUM_GRAPHS * _NCOMP
_CHUNK = 2560


def _sc_virial(n_edges, n_nodes):
    nsteps = _CHUNK // _L
    nch_tot = n_edges // _CHUNK
    trips = -(-nch_tot // _NW)
    mesh = plsc.VectorSubcoreMesh(core_axis_name="c", subcore_axis_name="s")

    @functools.partial(
        pl.kernel,
        out_type=jax.ShapeDtypeStruct((_NW, _ACC_ROWS), jnp.float32),
        mesh=mesh,
        compiler_params=pltpu.CompilerParams(needs_layout_passes=False),
        scratch_types=[
            pltpu.VMEM((n_nodes // 8,), jnp.int32),
            pltpu.VMEM((3, _CHUNK), jnp.float32),
            pltpu.VMEM((3, _CHUNK), jnp.float32),
            pltpu.VMEM((2, _CHUNK), jnp.int32),
            pltpu.VMEM((2, _CHUNK), jnp.int32),
            pltpu.VMEM((_ACC_ROWS, _L), jnp.float32),
            pltpu.VMEM((_ACC_ROWS, _L), jnp.float32),
            pltpu.VMEM((_ACC_ROWS, _L), jnp.float32),
            pltpu.VMEM((_ACC_ROWS, _L), jnp.float32),
            pltpu.VMEM((_ACC_ROWS,), jnp.float32),
            pltpu.SemaphoreType.DMA((2, 2)),
        ],
    )
    def k(disp_hbm, ei_hbm, batch_hbm, out_hbm, batch_v, dbuf0, dbuf1,
          ebuf0, ebuf1, acc_s0, acc_d0, acc_s1, acc_d1, res, sem):
        dbufs = (dbuf0, dbuf1)
        ebufs = (ebuf0, ebuf1)
        accs = (acc_s0, acc_d0, acc_s1, acc_d1)
        wid = lax.axis_index("s") * _NC + lax.axis_index("c")
        pltpu.sync_copy(batch_hbm, batch_v)
        zero = jnp.zeros((_L,), jnp.float32)
        for r in range(_ACC_ROWS):
            for a in accs:
                a[r, :] = zero
        lanes = lax.iota(jnp.int32, _L)

        def chof(t):
            return jnp.minimum(wid + t * _NW, nch_tot - 1)

        def start(t, slot):
            ch = chof(t)
            pltpu.make_async_copy(
                disp_hbm.at[:, pl.ds(ch * _CHUNK, _CHUNK)],
                dbufs[slot], sem.at[slot, 0]).start()
            pltpu.make_async_copy(
                ei_hbm.at[:, pl.ds(ch * _CHUNK, _CHUNK)],
                ebufs[slot], sem.at[slot, 1]).start()

        def wait(slot):
            pltpu.make_async_copy(
                disp_hbm.at[:, pl.ds(0, _CHUNK)],
                dbufs[slot], sem.at[slot, 0]).wait()
            pltpu.make_async_copy(
                ei_hbm.at[:, pl.ds(0, _CHUNK)],
                ebufs[slot], sem.at[slot, 1]).wait()

        def edge_vec(slot, off, acc_s, acc_d):
            dx = dbufs[slot][0, pl.ds(off, _L)]
            dy = dbufs[slot][1, pl.ds(off, _L)]
            dz = dbufs[slot][2, pl.ds(off, _L)]
            sv = ebufs[slot][0, pl.ds(off, _L)]
            tv = ebufs[slot][1, pl.ds(off, _L)]
            gsw = plsc.load_gather(
                batch_v, [lax.shift_right_logical(sv, 3)])
            gtw = plsc.load_gather(
                batch_v, [lax.shift_right_logical(tv, 3)])
            gs = lax.shift_right_logical(gsw, (sv & 7) * 4) & 15
            gt = lax.shift_right_logical(gtw, (tv & 7) * 4) & 15
            r2 = dx * dx + dy * dy + dz * dz + 1e-12
            bits = lax.bitcast_convert_type(r2, jnp.int32)
            bits = jnp.int32(0x5F3759DF) - lax.shift_right_arithmetic(bits, 1)
            y = lax.bitcast_convert_type(bits, jnp.float32)
            h = r2 * 0.5
            y = y * (1.5 - h * y * y)
            y = y * (1.5 - h * y * y)
            y = y * (1.5 - h * y * y)
            c = y - 1.0
            cx = c * dx
            cy = c * dy
            cz = c * dz
            vals = (cx * dx, cx * dy, cx * dz, cy * dy, cy * dz, cz * dz)
            rs = gs * _NCOMP
            rt = gt * _NCOMP
            for kk in range(_NCOMP):
                plsc.addupdate_scatter(acc_s, [rs + kk, lanes], vals[kk])
                plsc.addupdate_scatter(acc_d, [rt + kk, lanes], vals[kk])

        def compute(t, slot):
            @pl.when(wid + t * _NW < nch_tot)
            def _():
                @pl.loop(0, nsteps // 2)
                def _step(si):
                    off = si * (2 * _L)
                    edge_vec(slot, off, acc_s0, acc_d0)
                    edge_vec(slot, off + _L, acc_s1, acc_d1)

        start(0, 0)

        @pl.loop(0, trips // 2)
        def _t(t):
            k0 = 2 * t
            start(k0 + 1, 1)
            wait(0)
            compute(k0, 0)
            start(k0 + 2, 0)
            wait(1)
            compute(k0 + 1, 1)

        if trips % 2:
            wait(0)
            compute(trips - 1, 0)
        else:
            wait(0)
        for b in range(_ACC_ROWS // _L):
            rows = b * _L + lanes
            tot = zero
            for l in range(_L):
                col = jnp.full((_L,), l, dtype=jnp.int32)
                for a in accs:
                    tot = tot + plsc.load_gather(a, [rows, col])
            res[pl.ds(b * _L, _L)] = tot
        pltpu.sync_copy(res, out_hbm.at[wid])

    return k


def kernel(disp, edge_index, batch):
    n_edges = disp.shape[0]
    n_nodes = batch.shape[0]
    dflat = disp.astype(jnp.float32).T
    ei = edge_index.astype(jnp.int32)
    nib = batch.astype(jnp.int32).reshape(n_nodes // 8, 8)
    shifts = (jnp.arange(8, dtype=jnp.int32) * 4)[None, :]
    bat = jnp.sum(nib << shifts, axis=1, dtype=jnp.int32)
    part = _sc_virial(n_edges, n_nodes)(dflat, ei, bat)
    sym = jnp.sum(part, axis=0).reshape(_NUM_GRAPHS, _NCOMP)
    tri = jnp.array([[0, 1, 2], [1, 3, 4], [2, 4, 5]], dtype=jnp.int32)
    return sym[:, tri]


# speedup vs baseline: 1.5583x; 1.5583x over previous
import functools

import jax
import jax.numpy as jnp
from jax import lax
from jax.experimental import pallas as pl
from jax.experimental.pallas import tpu as pltpu
from jax.experimental.pallas import tpu_sc as plsc

_NUM_GRAPHS = 16
_NC = 2
_NS = 16
_NW = _NC * _NS
_L = 16
_NCOMP = 6
_ACC_ROWS = _NUM_GRAPHS * _NCOMP
_CHUNK = 2560


def _sc_virial(n_edges, n_nodes):
    nsteps = _CHUNK // _L
    nch_tot = n_edges // _CHUNK
    trips = -(-nch_tot // _NW)
    mesh = plsc.VectorSubcoreMesh(core_axis_name="c", subcore_axis_name="s")

    @functools.partial(
        pl.kernel,
        out_type=jax.ShapeDtypeStruct((_NW, _ACC_ROWS), jnp.float32),
        mesh=mesh,
        compiler_params=pltpu.CompilerParams(needs_layout_passes=False),
        scratch_types=[
            pltpu.VMEM((n_nodes // 8,), jnp.int32),
            pltpu.VMEM((3, _CHUNK), jnp.float32),
            pltpu.VMEM((3, _CHUNK), jnp.float32),
            pltpu.VMEM((2, _CHUNK), jnp.int32),
            pltpu.VMEM((2, _CHUNK), jnp.int32),
            pltpu.VMEM((_ACC_ROWS, _L), jnp.float32),
            pltpu.VMEM((_ACC_ROWS, _L), jnp.float32),
            pltpu.VMEM((_ACC_ROWS, _L), jnp.float32),
            pltpu.VMEM((_ACC_ROWS, _L), jnp.float32),
            pltpu.VMEM((_ACC_ROWS,), jnp.float32),
            pltpu.SemaphoreType.DMA((2, 2)),
        ],
    )
    def k(disp_hbm, ei_hbm, batch_hbm, out_hbm, batch_v, dbuf0, dbuf1,
          ebuf0, ebuf1, acc_s0, acc_d0, acc_s1, acc_d1, res, sem):
        dbufs = (dbuf0, dbuf1)
        ebufs = (ebuf0, ebuf1)
        accs = (acc_s0, acc_d0, acc_s1, acc_d1)
        wid = lax.axis_index("s") * _NC + lax.axis_index("c")
        pltpu.sync_copy(batch_hbm, batch_v)
        zero = jnp.zeros((_L,), jnp.float32)
        for r in range(_ACC_ROWS):
            for a in accs:
                a[r, :] = zero
        lanes = lax.iota(jnp.int32, _L)

        def chof(t):
            return jnp.minimum(wid + t * _NW, nch_tot - 1)

        def start(t, slot):
            ch = chof(t)
            pltpu.make_async_copy(
                disp_hbm.at[:, pl.ds(ch * _CHUNK, _CHUNK)],
                dbufs[slot], sem.at[slot, 0]).start()
            pltpu.make_async_copy(
                ei_hbm.at[:, pl.ds(ch * _CHUNK, _CHUNK)],
                ebufs[slot], sem.at[slot, 1]).start()

        def wait(slot):
            pltpu.make_async_copy(
                disp_hbm.at[:, pl.ds(0, _CHUNK)],
                dbufs[slot], sem.at[slot, 0]).wait()
            pltpu.make_async_copy(
                ei_hbm.at[:, pl.ds(0, _CHUNK)],
                ebufs[slot], sem.at[slot, 1]).wait()

        def edge_vec(slot, off, acc_s, acc_d):
            dx = dbufs[slot][0, pl.ds(off, _L)]
            dy = dbufs[slot][1, pl.ds(off, _L)]
            dz = dbufs[slot][2, pl.ds(off, _L)]
            sv = ebufs[slot][0, pl.ds(off, _L)]
            tv = ebufs[slot][1, pl.ds(off, _L)]
            gsw = plsc.load_gather(
                batch_v, [lax.shift_right_logical(sv, 3)])
            gtw = plsc.load_gather(
                batch_v, [lax.shift_right_logical(tv, 3)])
            gs = lax.shift_right_logical(gsw, (sv & 7) * 4) & 15
            gt = lax.shift_right_logical(gtw, (tv & 7) * 4) & 15
            r2 = dx * dx + dy * dy + dz * dz + 1e-12
            bits = lax.bitcast_convert_type(r2, jnp.int32)
            bits = jnp.int32(0x5F3759DF) - lax.shift_right_arithmetic(bits, 1)
            y = lax.bitcast_convert_type(bits, jnp.float32)
            h = r2 * 0.5
            y = y * (1.5 - h * y * y)
            y = y * (1.5 - h * y * y)
            y = y * (1.5 - h * y * y)
            c = y - 1.0
            cx = c * dx
            cy = c * dy
            cz = c * dz
            vals = (cx * dx, cx * dy, cx * dz, cy * dy, cy * dz, cz * dz)
            rs = gs * _NCOMP
            rt = gt * _NCOMP
            for kk in range(_NCOMP):
                plsc.addupdate_scatter(acc_s, [rs + kk, lanes], vals[kk])
                plsc.addupdate_scatter(acc_d, [rt + kk, lanes], vals[kk])

        def compute(t, slot):
            @pl.when(wid + t * _NW < nch_tot)
            def _():
                @plsc.parallel_loop(0, nsteps // 2)
                def _step(si):
                    off = si * (2 * _L)
                    edge_vec(slot, off, acc_s0, acc_d0)
                    edge_vec(slot, off + _L, acc_s1, acc_d1)

        start(0, 0)

        @pl.loop(0, trips // 2)
        def _t(t):
            k0 = 2 * t
            start(k0 + 1, 1)
            wait(0)
            compute(k0, 0)
            start(k0 + 2, 0)
            wait(1)
            compute(k0 + 1, 1)

        if trips % 2:
            wait(0)
            compute(trips - 1, 0)
        else:
            wait(0)
        for b in range(_ACC_ROWS // _L):
            rows = b * _L + lanes
            tot = zero
            for l in range(_L):
                col = jnp.full((_L,), l, dtype=jnp.int32)
                for a in accs:
                    tot = tot + plsc.load_gather(a, [rows, col])
            res[pl.ds(b * _L, _L)] = tot
        pltpu.sync_copy(res, out_hbm.at[wid])

    return k


def kernel(disp, edge_index, batch):
    n_edges = disp.shape[0]
    n_nodes = batch.shape[0]
    dflat = disp.astype(jnp.float32).T
    ei = edge_index.astype(jnp.int32)
    nib = batch.astype(jnp.int32).reshape(n_nodes // 8, 8)
    shifts = (jnp.arange(8, dtype=jnp.int32) * 4)[None, :]
    bat = jnp.sum(nib << shifts, axis=1, dtype=jnp.int32)
    part = _sc_virial(n_edges, n_nodes)(dflat, ei, bat)
    sym = jnp.sum(part, axis=0).reshape(_NUM_GRAPHS, _NCOMP)
    tri = jnp.array([[0, 1, 2], [1, 3, 4], [2, 4, 5]], dtype=jnp.int32)
    return sym[:, tri]


# speedup vs baseline: 1.8255x; 1.1714x over previous
import functools

import jax
import jax.numpy as jnp
from jax import lax
from jax.experimental import pallas as pl
from jax.experimental.pallas import tpu as pltpu
from jax.experimental.pallas import tpu_sc as plsc

_NUM_GRAPHS = 16
_NC = 2
_NS = 16
_NW = _NC * _NS
_L = 16
_NCOMP = 6
_ACC_ROWS = _NUM_GRAPHS * _NCOMP
_CHUNK = 2560


def _sc_virial(n_edges, n_nodes):
    nsteps = _CHUNK // _L
    nch_tot = n_edges // _CHUNK
    trips = -(-nch_tot // _NW)
    mesh = plsc.VectorSubcoreMesh(core_axis_name="c", subcore_axis_name="s")

    @functools.partial(
        pl.kernel,
        out_type=jax.ShapeDtypeStruct((_NW, _ACC_ROWS), jnp.float32),
        mesh=mesh,
        compiler_params=pltpu.CompilerParams(needs_layout_passes=False),
        scratch_types=[
            pltpu.VMEM((n_nodes // 8,), jnp.int32),
            pltpu.VMEM((3, _CHUNK), jnp.float32),
            pltpu.VMEM((3, _CHUNK), jnp.float32),
            pltpu.VMEM((2, _CHUNK), jnp.int32),
            pltpu.VMEM((2, _CHUNK), jnp.int32),
            pltpu.VMEM((_ACC_ROWS, _L), jnp.float32),
            pltpu.VMEM((_ACC_ROWS, _L), jnp.float32),
            pltpu.VMEM((_ACC_ROWS, _L), jnp.float32),
            pltpu.VMEM((_ACC_ROWS, _L), jnp.float32),
            pltpu.VMEM((_ACC_ROWS,), jnp.float32),
            pltpu.SemaphoreType.DMA((2, 2)),
        ],
    )
    def k(disp_hbm, ei_hbm, batch_hbm, out_hbm, batch_v, dbuf0, dbuf1,
          ebuf0, ebuf1, acc_s0, acc_d0, acc_s1, acc_d1, res, sem):
        dbufs = (dbuf0, dbuf1)
        ebufs = (ebuf0, ebuf1)
        accs = (acc_s0, acc_d0, acc_s1, acc_d1)
        wid = lax.axis_index("s") * _NC + lax.axis_index("c")
        pltpu.sync_copy(batch_hbm, batch_v)
        zero = jnp.zeros((_L,), jnp.float32)
        for r in range(_ACC_ROWS):
            for a in accs:
                a[r, :] = zero
        lanes = lax.iota(jnp.int32, _L)

        def chof(t):
            return jnp.minimum(wid + t * _NW, nch_tot - 1)

        def start(t, slot):
            ch = chof(t)
            pltpu.make_async_copy(
                disp_hbm.at[:, pl.ds(ch * _CHUNK, _CHUNK)],
                dbufs[slot], sem.at[slot, 0]).start()
            pltpu.make_async_copy(
                ei_hbm.at[:, pl.ds(ch * _CHUNK, _CHUNK)],
                ebufs[slot], sem.at[slot, 1]).start()

        def wait(slot):
            pltpu.make_async_copy(
                disp_hbm.at[:, pl.ds(0, _CHUNK)],
                dbufs[slot], sem.at[slot, 0]).wait()
            pltpu.make_async_copy(
                ei_hbm.at[:, pl.ds(0, _CHUNK)],
                ebufs[slot], sem.at[slot, 1]).wait()

        def edge_vec(slot, off, acc_s, acc_d):
            dx = dbufs[slot][0, pl.ds(off, _L)]
            dy = dbufs[slot][1, pl.ds(off, _L)]
            dz = dbufs[slot][2, pl.ds(off, _L)]
            sv = ebufs[slot][0, pl.ds(off, _L)]
            tv = ebufs[slot][1, pl.ds(off, _L)]
            gsw = plsc.load_gather(
                batch_v, [lax.shift_right_logical(sv, 3)])
            gtw = plsc.load_gather(
                batch_v, [lax.shift_right_logical(tv, 3)])
            gs = lax.shift_right_logical(gsw, (sv & 7) * 4) & 15
            gt = lax.shift_right_logical(gtw, (tv & 7) * 4) & 15
            r2 = dx * dx + dy * dy + dz * dz + 1e-12
            bits = lax.bitcast_convert_type(r2, jnp.int32)
            bits = jnp.int32(0x5F3759DF) - lax.shift_right_arithmetic(bits, 1)
            y = lax.bitcast_convert_type(bits, jnp.float32)
            h = r2 * 0.5
            y = y * (1.5 - h * y * y)
            y = y * (1.5 - h * y * y)
            y = y * (1.5 - h * y * y)
            c = y - 1.0
            cx = c * dx
            cy = c * dy
            cz = c * dz
            vals = (cx * dx, cx * dy, cx * dz, cy * dy, cy * dz, cz * dz)
            rs = gs * _NCOMP
            rt = gt * _NCOMP
            for kk in range(_NCOMP):
                plsc.addupdate_scatter(acc_s, [rs + kk, lanes], vals[kk])
                plsc.addupdate_scatter(acc_d, [rt + kk, lanes], vals[kk])

        def compute(t, slot):
            @pl.when(wid + t * _NW < nch_tot)
            def _():
                @plsc.parallel_loop(0, nsteps // 2, unroll=2)
                def _step(si):
                    off = si * (2 * _L)
                    edge_vec(slot, off, acc_s0, acc_d0)
                    edge_vec(slot, off + _L, acc_s1, acc_d1)

        start(0, 0)

        @pl.loop(0, trips // 2)
        def _t(t):
            k0 = 2 * t
            start(k0 + 1, 1)
            wait(0)
            compute(k0, 0)
            start(k0 + 2, 0)
            wait(1)
            compute(k0 + 1, 1)

        if trips % 2:
            wait(0)
            compute(trips - 1, 0)
        else:
            wait(0)
        for b in range(_ACC_ROWS // _L):
            rows = b * _L + lanes
            tot = zero
            for l in range(_L):
                col = jnp.full((_L,), l, dtype=jnp.int32)
                for a in accs:
                    tot = tot + plsc.load_gather(a, [rows, col])
            res[pl.ds(b * _L, _L)] = tot
        pltpu.sync_copy(res, out_hbm.at[wid])

    return k


def kernel(disp, edge_index, batch):
    n_edges = disp.shape[0]
    n_nodes = batch.shape[0]
    dflat = disp.astype(jnp.float32).T
    ei = edge_index.astype(jnp.int32)
    nib = batch.astype(jnp.int32).reshape(n_nodes // 8, 8)
    shifts = (jnp.arange(8, dtype=jnp.int32) * 4)[None, :]
    bat = jnp.sum(nib << shifts, axis=1, dtype=jnp.int32)
    part = _sc_virial(n_edges, n_nodes)(dflat, ei, bat)
    sym = jnp.sum(part, axis=0).reshape(_NUM_GRAPHS, _NCOMP)
    tri = jnp.array([[0, 1, 2], [1, 3, 4], [2, 4, 5]], dtype=jnp.int32)
    return sym[:, tri]
